# baseline (speedup 1.0000x reference)
import jax
import jax.numpy as jnp
from jax.experimental import pallas as pl
from jax.experimental.pallas import tpu as pltpu

NBR = 5
ROWS_PER_BLOCK = 1000


def _body(x_ref, n_ref, wb_ref, wl_ref, wr_ref, o_ref):
    agg = n_ref[:, 0, :] * wb_ref[0, :]
    for k in range(1, NBR):
        agg = agg + n_ref[:, k, :] * wb_ref[k, :]
    o_ref[...] = (
        jnp.dot(x_ref[...], wl_ref[...], preferred_element_type=jnp.float32)
        + jnp.dot(agg, wr_ref[...], preferred_element_type=jnp.float32)
    )


def kernel(x, neigh_x, w_aggr1, W_l, W_r):
    b, j, d = x.shape
    n_rows = b * j
    x2 = x.reshape(n_rows, d)
    n3 = neigh_x.reshape(n_rows, NBR, d)
    wb = jnp.pad(
        jnp.broadcast_to(w_aggr1[0][:, None], (NBR, d)), ((0, 8 - NBR), (0, 0))
    )
    wl_t = W_l.T
    wr_t = W_r.T

    r = ROWS_PER_BLOCK
    grid = (n_rows // r,)
    out = pl.pallas_call(
        _body,
        grid=grid,
        in_specs=[
            pl.BlockSpec((r, d), lambda i: (i, 0)),
            pl.BlockSpec((r, NBR, d), lambda i: (i, 0, 0)),
            pl.BlockSpec((8, d), lambda i: (0, 0)),
            pl.BlockSpec((d, d), lambda i: (0, 0)),
            pl.BlockSpec((d, d), lambda i: (0, 0)),
        ],
        out_specs=pl.BlockSpec((r, d), lambda i: (i, 0)),
        out_shape=jax.ShapeDtypeStruct((n_rows, d), jnp.float32),
        compiler_params=pltpu.CompilerParams(
            dimension_semantics=("arbitrary",),
        ),
    )(x2, n3, wb, wl_t, wr_t)
    return out


# speedup vs baseline: 1.5777x; 1.5777x over previous
import jax
import jax.numpy as jnp
from jax.experimental import pallas as pl
from jax.experimental.pallas import tpu as pltpu

NBR = 5
B_PER_BLOCK = 5


def _body(x_ref, n_ref, wb_ref, wl_ref, wr_ref, o_ref):
    bb, j, d = x_ref.shape
    r = bb * j
    agg = n_ref[:, :, 0, :] * wb_ref[0, :]
    for k in range(1, NBR):
        agg = agg + n_ref[:, :, k, :] * wb_ref[k, :]
    xb = x_ref[...].reshape(r, d)
    aggb = agg.reshape(r, d)
    o_ref[...] = (
        jnp.dot(xb, wl_ref[...], preferred_element_type=jnp.float32)
        + jnp.dot(aggb, wr_ref[...], preferred_element_type=jnp.float32)
    )


def kernel(x, neigh_x, w_aggr1, W_l, W_r):
    b, j, d = x.shape
    n_rows = b * j
    wb = jnp.pad(
        jnp.broadcast_to(w_aggr1[0][:, None], (NBR, d)), ((0, 8 - NBR), (0, 0))
    )
    wl_t = W_l.T
    wr_t = W_r.T

    bb = B_PER_BLOCK
    r = bb * j
    grid = (b // bb,)
    out = pl.pallas_call(
        _body,
        grid=grid,
        in_specs=[
            pl.BlockSpec((bb, j, d), lambda i: (i, 0, 0)),
            pl.BlockSpec((bb, j, NBR, d), lambda i: (i, 0, 0, 0)),
            pl.BlockSpec((8, d), lambda i: (0, 0)),
            pl.BlockSpec((d, d), lambda i: (0, 0)),
            pl.BlockSpec((d, d), lambda i: (0, 0)),
        ],
        out_specs=pl.BlockSpec((r, d), lambda i: (i, 0)),
        out_shape=jax.ShapeDtypeStruct((n_rows, d), jnp.float32),
        compiler_params=pltpu.CompilerParams(
            dimension_semantics=("arbitrary",),
        ),
    )(x, neigh_x, wb, wl_t, wr_t)
    return out


# speedup vs baseline: 1.6864x; 1.0689x over previous
import jax
import jax.numpy as jnp
from jax.experimental import pallas as pl
from jax.experimental.pallas import tpu as pltpu

NBR = 5
B_PER_BLOCK = 10


def _body(x_ref, n_ref, wb_ref, wl_ref, wr_ref, o_ref):
    bb, j, d = x_ref.shape
    r = bb * j
    agg = n_ref[:, :, 0, :] * wb_ref[0, :]
    for k in range(1, NBR):
        agg = agg + n_ref[:, :, k, :] * wb_ref[k, :]
    xb = x_ref[...].reshape(r, d)
    aggb = agg.reshape(r, d)
    o_ref[...] = (
        jnp.dot(xb, wl_ref[...], preferred_element_type=jnp.float32)
        + jnp.dot(aggb, wr_ref[...], preferred_element_type=jnp.float32)
    )


def kernel(x, neigh_x, w_aggr1, W_l, W_r):
    b, j, d = x.shape
    n_rows = b * j
    wb = jnp.pad(
        jnp.broadcast_to(w_aggr1[0][:, None], (NBR, d)), ((0, 8 - NBR), (0, 0))
    )
    wl_t = W_l.T
    wr_t = W_r.T

    bb = B_PER_BLOCK
    r = bb * j
    grid = (b // bb,)
    out = pl.pallas_call(
        _body,
        grid=grid,
        in_specs=[
            pl.BlockSpec((bb, j, d), lambda i: (i, 0, 0)),
            pl.BlockSpec((bb, j, NBR, d), lambda i: (i, 0, 0, 0)),
            pl.BlockSpec((8, d), lambda i: (0, 0)),
            pl.BlockSpec((d, d), lambda i: (0, 0)),
            pl.BlockSpec((d, d), lambda i: (0, 0)),
        ],
        out_specs=pl.BlockSpec((r, d), lambda i: (i, 0)),
        out_shape=jax.ShapeDtypeStruct((n_rows, d), jnp.float32),
        compiler_params=pltpu.CompilerParams(
            dimension_semantics=("arbitrary",),
        ),
    )(x, neigh_x, wb, wl_t, wr_t)
    return out


# speedup vs baseline: 1.7369x; 1.0300x over previous
import jax
import jax.numpy as jnp
from jax.experimental import pallas as pl
from jax.experimental.pallas import tpu as pltpu

NBR = 5
B_PER_BLOCK = 25


def _body(x_ref, n_ref, wb_ref, wl_ref, wr_ref, o_ref):
    bb, j, d = x_ref.shape
    r = bb * j
    agg = n_ref[:, :, 0, :] * wb_ref[0, :]
    for k in range(1, NBR):
        agg = agg + n_ref[:, :, k, :] * wb_ref[k, :]
    xb = x_ref[...].reshape(r, d)
    aggb = agg.reshape(r, d)
    o_ref[...] = (
        jnp.dot(xb, wl_ref[...], preferred_element_type=jnp.float32)
        + jnp.dot(aggb, wr_ref[...], preferred_element_type=jnp.float32)
    )


def kernel(x, neigh_x, w_aggr1, W_l, W_r):
    b, j, d = x.shape
    n_rows = b * j
    wb = jnp.pad(
        jnp.broadcast_to(w_aggr1[0][:, None], (NBR, d)), ((0, 8 - NBR), (0, 0))
    )
    wl_t = W_l.T
    wr_t = W_r.T

    bb = B_PER_BLOCK
    r = bb * j
    grid = (b // bb,)
    out = pl.pallas_call(
        _body,
        grid=grid,
        in_specs=[
            pl.BlockSpec((bb, j, d), lambda i: (i, 0, 0)),
            pl.BlockSpec((bb, j, NBR, d), lambda i: (i, 0, 0, 0)),
            pl.BlockSpec((8, d), lambda i: (0, 0)),
            pl.BlockSpec((d, d), lambda i: (0, 0)),
            pl.BlockSpec((d, d), lambda i: (0, 0)),
        ],
        out_specs=pl.BlockSpec((r, d), lambda i: (i, 0)),
        out_shape=jax.ShapeDtypeStruct((n_rows, d), jnp.float32),
        compiler_params=pltpu.CompilerParams(
            dimension_semantics=("arbitrary",),
        ),
    )(x, neigh_x, wb, wl_t, wr_t)
    return out


# speedup vs baseline: 1.7376x; 1.0004x over previous
import jax
import jax.numpy as jnp
from jax.experimental import pallas as pl
from jax.experimental.pallas import tpu as pltpu

NBR = 5
B_PER_BLOCK = 25


def _body(x_ref, n_ref, wb_ref, wl_ref, wr_ref, o_ref):
    bb, j, d = x_ref.shape
    r = bb * j
    agg = n_ref[:, :, 0, :] * wb_ref[0, :]
    for k in range(1, NBR):
        agg = agg + n_ref[:, :, k, :] * wb_ref[k, :]
    xb = x_ref[...].reshape(r, d)
    aggb = agg.reshape(r, d)
    o_ref[...] = (
        jnp.dot(xb, wl_ref[...], preferred_element_type=jnp.float32)
        + jnp.dot(aggb, wr_ref[...], preferred_element_type=jnp.float32)
    )


def kernel(x, neigh_x, w_aggr1, W_l, W_r):
    b, j, d = x.shape
    n_rows = b * j
    wb = jnp.pad(
        jnp.broadcast_to(w_aggr1[0][:, None], (NBR, d)), ((0, 8 - NBR), (0, 0))
    )
    wl_t = W_l.T
    wr_t = W_r.T

    bb = B_PER_BLOCK
    r = bb * j
    grid = (b // bb,)
    out = pl.pallas_call(
        _body,
        grid=grid,
        in_specs=[
            pl.BlockSpec((bb, j, d), lambda i: (i, 0, 0)),
            pl.BlockSpec((bb, j, NBR, d), lambda i: (i, 0, 0, 0)),
            pl.BlockSpec((8, d), lambda i: (0, 0)),
            pl.BlockSpec((d, d), lambda i: (0, 0)),
            pl.BlockSpec((d, d), lambda i: (0, 0)),
        ],
        out_specs=pl.BlockSpec((r, d), lambda i: (i, 0)),
        out_shape=jax.ShapeDtypeStruct((n_rows, d), jnp.float32),
        compiler_params=pltpu.CompilerParams(
            dimension_semantics=("parallel",),
        ),
    )(x, neigh_x, wb, wl_t, wr_t)
    return out
